# baseline (speedup 1.0000x reference)
import functools

import jax
import jax.numpy as jnp
from jax import lax
from jax.experimental import pallas as pl
from jax.experimental.pallas import tpu as pltpu
from jax.experimental.pallas import tpu_sc as plsc

NUM_VIRTUAL_TOKENS = 50
TOKEN_DIM = 1024
EMB_DIM = 24 * 2 * TOKEN_DIM
BATCH = 32
NPOS = BATCH * NUM_VIRTUAL_TOKENS
NTILE = 32
CHUNK = EMB_DIM // NTILE
NCHUNKS = NPOS // 16


def _body(prefix_hbm, emb_hbm, out_hbm, idx_v, cache_v, dsem):
  c = lax.axis_index("c")
  s = lax.axis_index("s")
  t = s * 2 + c

  pltpu.sync_copy(emb_hbm.at[t], cache_v)
  pltpu.sync_copy(prefix_hbm, idx_v)

  def drain_one():
    pltpu.make_async_copy(cache_v.at[pl.ds(0, 1)],
                          out_hbm.at[pl.ds(0, 1), t], dsem).wait()

  def chunk_body(cc, carry):
    off = pl.multiple_of(cc * 16, 16)
    vals = idx_v[pl.ds(off, 16)]
    for l in range(16):
      vidx = vals[l]
      pltpu.async_copy(cache_v.at[pl.ds(vidx, 1)],
                       out_hbm.at[pl.ds(off + l, 1), t], dsem)

    @pl.when(cc >= 1)
    def _():
      for _ in range(16):
        drain_one()

    return carry

  lax.fori_loop(0, NCHUNKS, chunk_body, 0)
  for _ in range(16):
    drain_one()


@jax.jit
def kernel(prefix, embedding):
  emb_t = jnp.transpose(
      embedding.reshape(NUM_VIRTUAL_TOKENS, NTILE, CHUNK), (1, 0, 2))
  mesh = plsc.VectorSubcoreMesh(core_axis_name="c", subcore_axis_name="s")
  k = functools.partial(
      pl.kernel,
      out_type=jax.ShapeDtypeStruct((NPOS, NTILE, CHUNK), jnp.float32),
      mesh=mesh,
      scratch_types=[
          pltpu.VMEM((NPOS,), jnp.int32),
          pltpu.VMEM((NUM_VIRTUAL_TOKENS, CHUNK), jnp.float32),
          pltpu.SemaphoreType.DMA,
      ],
  )(_body)
  out = k(prefix.reshape(NPOS), emb_t)
  return out.reshape(BATCH, NUM_VIRTUAL_TOKENS, EMB_DIM)


# speedup vs baseline: 1.8110x; 1.8110x over previous
import functools

import jax
import jax.numpy as jnp
from jax import lax
from jax.experimental import pallas as pl
from jax.experimental.pallas import tpu as pltpu
from jax.experimental.pallas import tpu_sc as plsc

NUM_VIRTUAL_TOKENS = 50
TOKEN_DIM = 1024
EMB_DIM = 24 * 2 * TOKEN_DIM
BATCH = 32


def _body(prefix_hbm, emb_hbm, out_hbm, idx_v, buf0, buf1, gsem0, gsem1,
          ssem0, ssem1):
  c = lax.axis_index("c")
  s = lax.axis_index("s")
  w = s * 2 + c

  pltpu.sync_copy(prefix_hbm.at[w], idx_v)

  def g_start(i, buf, gsem):
    pltpu.async_copy(emb_hbm.at[idx_v.at[i]], buf, gsem)

  def g_wait(buf, gsem):
    pltpu.make_async_copy(emb_hbm.at[idx_v.at[0]], buf, gsem).wait()

  def s_start(i, buf, ssem):
    pltpu.async_copy(buf, out_hbm.at[w, pl.ds(i, 1)], ssem)

  def s_wait(i, buf, ssem):
    pltpu.make_async_copy(buf, out_hbm.at[w, pl.ds(i, 1)], ssem).wait()

  bufs = (buf0, buf1)
  gsems = (gsem0, gsem1)
  ssems = (ssem0, ssem1)

  g_start(0, buf0, gsem0)
  g_start(1, buf1, gsem1)

  def j_body(j, carry):
    for b in range(2):
      i = 2 * j + b
      g_wait(bufs[b], gsems[b])
      s_start(i, bufs[b], ssems[b])
      s_wait(i, bufs[b], ssems[b])

      @pl.when(i + 2 < NUM_VIRTUAL_TOKENS)
      def _():
        g_start(i + 2, bufs[b], gsems[b])

    return carry

  lax.fori_loop(0, NUM_VIRTUAL_TOKENS // 2, j_body, 0)


@jax.jit
def kernel(prefix, embedding):
  mesh = plsc.VectorSubcoreMesh(core_axis_name="c", subcore_axis_name="s")
  k = functools.partial(
      pl.kernel,
      out_type=jax.ShapeDtypeStruct((BATCH, NUM_VIRTUAL_TOKENS, EMB_DIM),
                                    jnp.float32),
      mesh=mesh,
      scratch_types=[
          pltpu.VMEM((NUM_VIRTUAL_TOKENS, 1), jnp.int32),
          pltpu.VMEM((1, EMB_DIM), jnp.float32),
          pltpu.VMEM((1, EMB_DIM), jnp.float32),
          pltpu.SemaphoreType.DMA,
          pltpu.SemaphoreType.DMA,
          pltpu.SemaphoreType.DMA,
          pltpu.SemaphoreType.DMA,
      ],
  )(_body)
  return k(prefix.reshape(BATCH, NUM_VIRTUAL_TOKENS, 1), embedding)
